# Initial kernel scaffold; baseline (speedup 1.0000x reference)
import jax
import jax.numpy as jnp
from jax.experimental import pallas as pl


def kernel(f, bases_c, bases_s, bases_0, nodes, normal_vectors, directed_edges, node_weights, weights_c, weights_s, weights_0, w_weight, w_bias):
    raise NotImplementedError("write your pallas kernel here")


# speedup vs baseline: 31.4643x; 31.4643x over previous
import functools

import jax
import jax.numpy as jnp
from jax import lax
from jax.experimental import pallas as pl
from jax.experimental.pallas import tpu as pltpu
from jax.experimental.pallas import tpu_sc as plsc

NC = 2
NS = 16
NWK = NC * NS
L = 16
CH = 128
EB = 8

K = 16
C = 32
TREC_W = 64
SREC_W = 64
SREC2_W = 16


def _sc_body(n_chunks, rows_per_sub,
             tgt_hbm, src_hbm, nw_hbm, trec_hbm, srec_hbm, srec2_hbm,
             wct_hbm, wst_hbm, w0_hbm, zeros_hbm,
             fout_hbm,
             fout_sh, tgt_v, src_v, nw_v, trec_b, srec_b, srec2_b,
             wct_v, wst_v, w0_v, msg_b,
             sem_t, sem_s, sem_s2):
    cid = lax.axis_index("c")
    sid = lax.axis_index("s")
    wid = sid * NC + cid
    epw = n_chunks * CH

    r0 = sid * rows_per_sub
    pltpu.sync_copy(zeros_hbm.at[pl.ds(r0, rows_per_sub)],
                    fout_sh.at[pl.ds(r0, rows_per_sub)])

    pltpu.sync_copy(wct_hbm, wct_v)
    pltpu.sync_copy(wst_hbm, wst_v)
    pltpu.sync_copy(w0_hbm, w0_v)

    plsc.subcore_barrier()

    ebase0 = wid * epw

    def chunk_body(i, carry):
        ebase = ebase0 + i * CH
        pltpu.sync_copy(tgt_hbm.at[pl.ds(ebase, CH)], tgt_v)
        pltpu.sync_copy(src_hbm.at[pl.ds(ebase, CH)], src_v)
        pltpu.sync_copy(nw_hbm.at[pl.ds(ebase, CH)], nw_v)
        ct = pltpu.async_copy(trec_hbm.at[tgt_v], trec_b, sem_t)
        cs = pltpu.async_copy(srec_hbm.at[src_v], srec_b, sem_s)
        cs2 = pltpu.async_copy(srec2_hbm.at[src_v], srec2_b, sem_s2)
        ct.wait()
        cs.wait()
        cs2.wait()

        w0l = w0_v[pl.ds(0, L)]
        w0h = w0_v[pl.ds(L, L)]

        def sblock_body(b, carry2):
            e0 = b * L
            nw16 = nw_v[pl.ds(e0, L)]
            for h in range(2):
                pcs, pss, avs = [], [], []
                for j in range(EB):
                    e = e0 + h * EB + j
                    thead = trec_b[e, pl.ds(0, L)]
                    shead = srec_b[e, pl.ds(0, L)]
                    bct = trec_b[e, pl.ds(2, L)]
                    bst = trec_b[e, pl.ds(18, L)]
                    bcs = srec_b[e, pl.ds(4, L)]
                    bss = srec_b[e, pl.ds(20, L)]
                    pcs.append(bct * bcs + bst * bss)
                    pss.append(bct * bss - bst * bcs)
                    dx = thead[0] - shead[0]
                    dy = thead[1] - shead[1]
                    r2 = dx * dx + dy * dy + 1e-6
                    num = dx * shead[2] + dy * shead[3]
                    glv = jnp.full((L,), num) / jnp.full((L,), r2)
                    avs.append(glv * nw16[h * EB + j])
                acc_l = [w0l] * EB
                acc_h = [w0h] * EB
                for k in range(K):
                    wcl = wct_v[pl.ds(k * C, L)]
                    wch = wct_v[pl.ds(k * C + L, L)]
                    wsl = wst_v[pl.ds(k * C, L)]
                    wsh = wst_v[pl.ds(k * C + L, L)]
                    for j in range(EB):
                        pck = pcs[j][k]
                        psk = pss[j][k]
                        acc_l[j] = acc_l[j] + pck * wcl + psk * wsl
                        acc_h[j] = acc_h[j] + pck * wch + psk * wsh
                for j in range(EB):
                    e = e0 + h * EB + j
                    fsl = srec_b[e, pl.ds(36, L)]
                    fsh = srec2_b[e, pl.ds(0, L)]
                    msg_b[e, pl.ds(0, L)] = (avs[j] * fsl) * acc_l[j]
                    msg_b[e, pl.ds(L, L)] = (avs[j] * fsh) * acc_h[j]
            return carry2

        lax.fori_loop(0, CH // L, sblock_body, 0)

        pltpu.sync_copy(msg_b, fout_sh.at[tgt_v], add=True)
        return carry

    lax.fori_loop(0, n_chunks, chunk_body, 0)

    plsc.subcore_barrier()

    pltpu.sync_copy(fout_sh.at[pl.ds(r0, rows_per_sub)],
                    fout_hbm.at[cid, pl.ds(r0, rows_per_sub)])


def _sc_edge_call(N_pad, n_chunks, tgt_p, src_p, nw_p, trec, srec, srec2,
                  wct, wst, w0p, zeros_nc):
    rows_per_sub = N_pad // NS
    mesh = plsc.VectorSubcoreMesh(core_axis_name="c", subcore_axis_name="s")
    body = functools.partial(_sc_body, n_chunks, rows_per_sub)
    return pl.kernel(
        body,
        out_type=jax.ShapeDtypeStruct((NC, N_pad, C), jnp.float32),
        mesh=mesh,
        compiler_params=pltpu.CompilerParams(use_tc_tiling_on_sc=False),
        scratch_types=[
            pltpu.VMEM_SHARED((N_pad, C), jnp.float32),
            pltpu.VMEM((CH,), jnp.int32),
            pltpu.VMEM((CH,), jnp.int32),
            pltpu.VMEM((CH,), jnp.float32),
            pltpu.VMEM((CH, TREC_W), jnp.float32),
            pltpu.VMEM((CH, SREC_W), jnp.float32),
            pltpu.VMEM((CH, SREC2_W), jnp.float32),
            pltpu.VMEM((K * C,), jnp.float32),
            pltpu.VMEM((K * C,), jnp.float32),
            pltpu.VMEM((C,), jnp.float32),
            pltpu.VMEM((CH, C), jnp.float32),
            pltpu.SemaphoreType.DMA,
            pltpu.SemaphoreType.DMA,
            pltpu.SemaphoreType.DMA,
        ],
    )(tgt_p, src_p, nw_p, trec, srec, srec2, wct, wst, w0p, zeros_nc)


def _proj_body(w_ref, b_ref, fa_ref, fb_ref, o_ref):
    x = fa_ref[...] + fb_ref[...]
    mm = lax.dot_general(w_ref[...], x, (((1,), (1,)), ((), ())),
                         preferred_element_type=jnp.float32)
    o_ref[...] = mm + b_ref[...]


def _proj_call(N, w2d, b2d, fa, fb):
    BN = 512
    nb = (N + BN - 1) // BN
    return pl.pallas_call(
        _proj_body,
        out_shape=jax.ShapeDtypeStruct((C, N), jnp.float32),
        grid=(nb,),
        in_specs=[
            pl.BlockSpec((C, C), lambda i: (0, 0)),
            pl.BlockSpec((C, 1), lambda i: (0, 0)),
            pl.BlockSpec((BN, C), lambda i: (i, 0)),
            pl.BlockSpec((BN, C), lambda i: (i, 0)),
        ],
        out_specs=pl.BlockSpec((C, BN), lambda i: (0, i)),
    )(w2d, b2d, fa, fb)


def kernel(f, bases_c, bases_s, bases_0, nodes, normal_vectors, directed_edges,
           node_weights, weights_c, weights_s, weights_0, w_weight, w_bias):
    del bases_0
    B, C_in, N = f.shape
    E = directed_edges.shape[1]
    f32 = jnp.float32

    fp = jnp.transpose(f[0])
    bc = bases_c[0, :, :, 0]
    bs = bases_s[0, :, :, 0]
    nod = nodes[0]
    nv = normal_vectors[0]
    tgt = directed_edges[0, :, 0, 0]
    src = directed_edges[0, :, 1, 0]
    nw = node_weights[0, :, 0]

    trec = jnp.concatenate(
        [nod, bc, bs, jnp.zeros((N, TREC_W - 2 - 2 * K), f32)], axis=1)
    srec = jnp.concatenate(
        [nod, nv, bc, bs, fp[:, :L],
         jnp.zeros((N, SREC_W - 4 - 2 * K - L), f32)], axis=1)
    srec2 = fp[:, L:]

    wct = jnp.reshape(jnp.transpose(2.0 * weights_c[:, :, 0]), (-1,))
    wst = jnp.reshape(jnp.transpose(2.0 * weights_s[:, :, 0]), (-1,))
    w0p = 1.0 + weights_0[:, 0, 0]

    quantum = NWK * CH
    E_pad = ((E + quantum - 1) // quantum) * quantum
    pad = E_pad - E
    tgt_p = jnp.concatenate([tgt, jnp.zeros((pad,), jnp.int32)])
    src_p = jnp.concatenate([src, jnp.zeros((pad,), jnp.int32)])
    nw_p = jnp.concatenate([nw, jnp.zeros((pad,), f32)])
    n_chunks = E_pad // quantum

    N_pad = ((N + 8 * NS - 1) // (8 * NS)) * (8 * NS)
    zeros_nc = jnp.zeros((N_pad, C), f32)
    fout2 = _sc_edge_call(N_pad, n_chunks, tgt_p, src_p, nw_p, trec, srec,
                          srec2, wct, wst, w0p, zeros_nc)

    out = _proj_call(N_pad, w_weight[:, :, 0], w_bias[:, None],
                     fout2[0], fout2[1])
    return out[None, :, :N]
